# baseline (speedup 1.0000x reference)
import functools

import jax
import jax.numpy as jnp
from jax import lax
from jax.experimental import pallas as pl
from jax.experimental.pallas import tpu as pltpu
from jax.experimental.pallas import tpu_sc as plsc

N = 10000
D = 128
OUT = 128
E = 320000
NETS = 2
ALPHA = 0.5

NT = 32
NSUB = 16
PT = E // NT
BB = 50
NB = (E // NSUB) // BB
W144 = 144
NP = 10240
RPT = NP // NSUB
ET = E // NSUB
CH = 20
RB = 1000
GB = N // RB


def _sc_spmm(tbl, gsrc, gdst, z144):
    mesh = plsc.VectorSubcoreMesh(core_axis_name="c", subcore_axis_name="s")
    half = NB // CH // 2

    @functools.partial(
        pl.kernel,
        out_type=jax.ShapeDtypeStruct((4, 2, NP, W144), jnp.float32),
        mesh=mesh,
        compiler_params=pltpu.CompilerParams(use_tc_tiling_on_sc=False),
        scratch_types=[
            pltpu.VMEM_SHARED((NP, W144), jnp.float32),
            pltpu.VMEM((CH, BB), jnp.int32),
            pltpu.VMEM((CH, BB), jnp.int32),
            pltpu.VMEM((CH, BB), jnp.int32),
            pltpu.VMEM((CH, BB), jnp.int32),
            pltpu.VMEM((BB, W144), jnp.float32),
            pltpu.VMEM((BB, W144), jnp.float32),
            pltpu.VMEM((BB, W144), jnp.float32),
            pltpu.VMEM((BB, W144), jnp.float32),
            pltpu.SemaphoreType.DMA,
            pltpu.SemaphoreType.DMA,
            pltpu.SemaphoreType.DMA,
            pltpu.SemaphoreType.DMA,
            pltpu.SemaphoreType.DMA,
            pltpu.SemaphoreType.DMA,
            pltpu.SemaphoreType.DMA,
            pltpu.SemaphoreType.DMA,
            pltpu.SemaphoreType.DMA,
            pltpu.SemaphoreType.DMA,
            pltpu.SemaphoreType.DMA,
            pltpu.SemaphoreType.DMA,
        ],
    )
    def k(tbl_h, gsrc_h, gdst_h, z144_h, out_h,
          acc, src_a, src_b, dst_a, dst_b, b0, b1, b2, b3,
          sg0, sg1, sg2, sg3, st0, st1, st2, st3, ss_a, sd_a, ss_b, sd_b):
        cid = lax.axis_index("c")
        sid = lax.axis_index("s")
        row0 = sid * RPT
        bufs = (b0, b1, b2, b3)
        semg = (sg0, sg1, sg2, sg3)
        sems = (st0, st1, st2, st3)
        for p in range(4):
            tblp = tbl_h.at[cid, p]
            gs = gsrc_h.at[p, sid]
            gd = gdst_h.at[p, sid]
            pltpu.sync_copy(z144_h, acc.at[pl.ds(row0, RPT)])
            pltpu.sync_copy(gs.at[pl.ds(0, CH)], src_a)
            pltpu.sync_copy(gd.at[pl.ds(0, CH)], dst_a)
            pltpu.async_copy(gs.at[pl.ds(CH, CH)], src_b, ss_b)
            pltpu.async_copy(gd.at[pl.ds(CH, CH)], dst_b, sd_b)
            plsc.subcore_barrier()
            pltpu.async_copy(tblp.at[src_a.at[0]], b0, sg0)
            pltpu.async_copy(tblp.at[src_a.at[1]], b1, sg1)

            def step(q, j, src_c, dst_c, src_n, ss_n, first_pair,
                     tblp, gs):
                k4 = j % 4
                kk = (j + 2) % 4
                pltpu.make_async_copy(tblp.at[src_c.at[j]], bufs[k4],
                                      semg[k4]).wait()
                pltpu.async_copy(bufs[k4], acc.at[dst_c.at[j]], sems[k4],
                                 add=True)
                if first_pair and j < 2:
                    @pl.when(q > 0)
                    def _():
                        pltpu.make_async_copy(bufs[kk], acc.at[pl.ds(0, BB)],
                                              sems[kk]).wait()
                else:
                    pltpu.make_async_copy(bufs[kk], acc.at[pl.ds(0, BB)],
                                          sems[kk]).wait()
                if j <= CH - 3:
                    pltpu.async_copy(tblp.at[src_c.at[j + 2]], bufs[kk],
                                     semg[kk])
                elif j == CH - 2:
                    if ss_n is None:
                        @pl.when(q < half - 1)
                        def _():
                            pltpu.make_async_copy(gs.at[pl.ds(0, CH)], src_n,
                                                  ss_a).wait()
                            pltpu.async_copy(tblp.at[src_n.at[0]], bufs[kk],
                                             semg[kk])
                    else:
                        pltpu.make_async_copy(gs.at[pl.ds(0, CH)], src_n,
                                              ss_n).wait()
                        pltpu.async_copy(tblp.at[src_n.at[0]], bufs[kk],
                                         semg[kk])
                else:
                    if ss_n is None:
                        @pl.when(q < half - 1)
                        def _():
                            pltpu.async_copy(tblp.at[src_n.at[1]], bufs[kk],
                                             semg[kk])
                    else:
                        pltpu.async_copy(tblp.at[src_n.at[1]], bufs[kk],
                                         semg[kk])

            def body(q, c, tblp=tblp, gs=gs, gd=gd):
                @pl.when(q > 0)
                def _():
                    pltpu.make_async_copy(gd.at[pl.ds(0, CH)], dst_a,
                                          sd_a).wait()
                for j in range(CH):
                    step(q, j, src_a, dst_a, src_b, ss_b, True, tblp, gs)

                @pl.when(q < half - 1)
                def _():
                    pltpu.async_copy(gs.at[pl.ds((q + 1) * 2 * CH, CH)],
                                     src_a, ss_a)
                    pltpu.async_copy(gd.at[pl.ds((q + 1) * 2 * CH, CH)],
                                     dst_a, sd_a)

                pltpu.make_async_copy(gd.at[pl.ds(0, CH)], dst_b, sd_b).wait()
                for j in range(CH):
                    step(q, j, src_b, dst_b, src_a, None, False, tblp, gs)

                @pl.when(q < half - 1)
                def _():
                    pltpu.async_copy(gs.at[pl.ds((q + 1) * 2 * CH + CH, CH)],
                                     src_b, ss_b)
                    pltpu.async_copy(gd.at[pl.ds((q + 1) * 2 * CH + CH, CH)],
                                     dst_b, sd_b)
                return c

            lax.fori_loop(0, half, body, 0)
            pltpu.make_async_copy(b2, acc.at[pl.ds(0, BB)], st2).wait()
            pltpu.make_async_copy(b3, acc.at[pl.ds(0, BB)], st3).wait()
            plsc.subcore_barrier()
            pltpu.sync_copy(acc.at[pl.ds(row0, RPT)],
                            out_h.at[p, cid, pl.ds(row0, RPT)])

    return k(tbl, gsrc, gdst, z144)


def _prep_body(x_r, id_r, od_r, wo_r, wi_r, ei_r, eo_r,
               tbl_r, aux_r):
    To = eo_r.shape[0]
    Ti = ei_r.shape[0]
    xb = x_r[...]
    r = 1.0 / (jnp.sum(jnp.abs(xb), axis=1, keepdims=True) + 1e-12)
    x1 = xb * r
    x1sq = x1 * x1
    wo = wo_r[...]
    wi = wi_r[...]
    s_out = jnp.sum(x1sq * wo, axis=1, keepdims=True)
    s_in = jnp.sum(x1sq * wi, axis=1, keepdims=True)
    embw_o = jnp.sum(eo_r[...] * wo, axis=1)
    embw_i = jnp.sum(ei_r[...] * wi, axis=1)
    z15 = jnp.zeros((RB, 15), jnp.float32)
    z16 = jnp.zeros((RB, 16), jnp.float32)
    cols = [s_out, s_in]
    invs = []
    for i in range(NETS):
        od = od_r[...][:, i]
        idg = id_r[...][:, i]
        oinv = jnp.where(od > 0,
                         lax.rsqrt(jnp.maximum(od.astype(jnp.float32), 1.0)),
                         0.0)
        iinv = jnp.where(idg > 0,
                         lax.rsqrt(jnp.maximum(idg.astype(jnp.float32), 1.0)),
                         0.0)
        ooh = od[:, None] == lax.broadcasted_iota(jnp.int32, (RB, To), 1)
        eo_g = jnp.sum(jnp.where(ooh, embw_o[None, :], 0.0), axis=1,
                       keepdims=True)
        ioh = idg[:, None] == lax.broadcasted_iota(jnp.int32, (RB, Ti), 1)
        ei_g = jnp.sum(jnp.where(ioh, embw_i[None, :], 0.0), axis=1,
                       keepdims=True)
        oinv = oinv[:, None]
        iinv = iinv[:, None]
        tbl_r[0, 2 * i] = jnp.concatenate([iinv * xb, iinv * s_out, z15],
                                          axis=1)
        tbl_r[0, 2 * i + 1] = jnp.concatenate([oinv * xb, oinv * s_in, z15],
                                              axis=1)
        tbl_r[1, 2 * i] = jnp.concatenate([iinv * x1, z16], axis=1)
        tbl_r[1, 2 * i + 1] = jnp.concatenate([oinv * x1, z16], axis=1)
        cols.append(eo_g)
        cols.append(ei_g)
        invs.append((oinv, iinv))
    aux_r[...] = jnp.concatenate(
        cols + [invs[0][0], invs[1][0], invs[0][1], invs[1][1],
                jnp.zeros((RB, 6), jnp.float32)], axis=1)


def _prep(x, ideg_t, odeg_t, w_out_f, w_in_f, emb_in, emb_out):
    To = emb_out.shape[0]
    Ti = emb_in.shape[0]
    return pl.pallas_call(
        _prep_body,
        grid=(GB,),
        in_specs=[
            pl.BlockSpec((RB, D), lambda b: (b, 0)),
            pl.BlockSpec((RB, NETS), lambda b: (b, 0)),
            pl.BlockSpec((RB, NETS), lambda b: (b, 0)),
            pl.BlockSpec((1, D), lambda b: (0, 0)),
            pl.BlockSpec((1, D), lambda b: (0, 0)),
            pl.BlockSpec((Ti, D), lambda b: (0, 0)),
            pl.BlockSpec((To, D), lambda b: (0, 0)),
        ],
        out_specs=[
            pl.BlockSpec((2, 4, RB, W144), lambda b: (0, 0, b, 0)),
            pl.BlockSpec((RB, 16), lambda b: (b, 0)),
        ],
        out_shape=[
            jax.ShapeDtypeStruct((2, 4, N, W144), jnp.float32),
            jax.ShapeDtypeStruct((N, 16), jnp.float32),
        ],
    )(x, ideg_t, odeg_t, w_out_f, w_in_f, emb_in, emb_out)


def _post_body(ox_r, aux_r, x_r, omt_r, ombt_r, imt_r, imbt_r,
               ws_r, bs_r, wd_r, bd_r, wf_r, bf_r, wo_r, wi_r, bof_r, bif_r,
               out_r, cin_r, cout_r, bn_r):
    pid = pl.program_id(0)
    xb = x_r[...]
    r = 1.0 / (jnp.sum(jnp.abs(xb), axis=1, keepdims=True) + 1e-12)
    x1 = xb * r
    aux = aux_r[...]
    gid = pid * RB + lax.broadcasted_iota(jnp.int32, (RB, 1), 0)
    dfac = jnp.where(gid == N - 1, 2.0, 1.0)
    wo = wo_r[...]
    wi = wi_r[...]
    tau = jnp.exp(jnp.float32(0.0)) + jnp.float32(0.1)
    acc_src = jnp.zeros((RB, D), jnp.float32)
    acc_dst = jnp.zeros((RB, D), jnp.float32)
    s_co = jnp.zeros((RB, 1), jnp.float32)
    s_ci = jnp.zeros((RB, 1), jnp.float32)
    for i in range(NETS):
        oinv = aux[:, 6 + i:7 + i]
        iinv = aux[:, 8 + i:9 + i]
        fa = ox_r[2 * i, 0]
        fb = ox_r[2 * i, 1]
        ga = ox_r[2 * i + 1, 0]
        gb = ox_r[2 * i + 1, 1]
        out_nei = fa[:, 0:D] * oinv
        mvs_o = fa[:, D:D + 1] * oinv
        mvx1_o = fb[:, 0:D] * oinv
        in_nei = ga[:, 0:D] * iinv
        mvs_i = ga[:, D:D + 1] * iinv
        mvx1_i = gb[:, 0:D] * iinv
        eow = (aux[:, 0:1] * dfac + mvs_o
               - 2.0 * jnp.sum(mvx1_o * x1 * wo, axis=1, keepdims=True))
        eiw = (aux[:, 1:2] * dfac + mvs_i
               - 2.0 * jnp.sum(mvx1_i * x1 * wi, axis=1, keepdims=True))
        co_l = -eow + aux[:, 2 + 2 * i:3 + 2 * i] + bof_r[0, 0]
        ci_l = -eiw + aux[:, 3 + 2 * i:4 + 2 * i] + bif_r[0, 0]
        co = 1.0 / (1.0 + jnp.exp((ci_l - co_l) / tau))
        ci = 1.0 - co
        co = co * omt_r[...][:, i:i + 1] + ombt_r[...][:, i:i + 1]
        ci = ci * imt_r[...][:, i:i + 1] + imbt_r[...][:, i:i + 1]
        acc_src += co * out_nei
        acc_dst += ci * in_nei
        s_co += co
        s_ci += ci
    dn = (((1,), (1,)), ((), ()))
    outv = (lax.dot_general(acc_src, ws_r[...], dn,
                            preferred_element_type=jnp.float32)
            + s_co * bs_r[...]
            + lax.dot_general(acc_dst, wd_r[...], dn,
                              preferred_element_type=jnp.float32)
            + s_ci * bd_r[...]) * jnp.float32(1.0 / NETS)
    outv = outv + ALPHA * (lax.dot_general(xb, wf_r[...], dn,
                                           preferred_element_type=jnp.float32)
                           + bf_r[...])
    out_r[...] = outv
    cout_r[...] = s_co * jnp.float32(1.0 / NETS)
    cin_r[...] = s_ci * jnp.float32(1.0 / NETS)

    @pl.when(pid == 0)
    def _():
        bn_r[...] = jnp.zeros((8, D), jnp.float32)

    bn_r[0:1, :] += jnp.sum(outv, axis=0, keepdims=True)
    bn_r[1:2, :] += jnp.sum(outv * outv, axis=0, keepdims=True)


def _post(ox, aux, x, omt, ombt, imt, imbt,
          w_src, b_src, w_dst, b_dst, w_fc, b_fc, w_out_f, w_in_f,
          b_out_f, b_in_f):
    full = lambda s: pl.BlockSpec(s, lambda b: tuple(0 for _ in s))
    return pl.pallas_call(
        _post_body,
        grid=(GB,),
        in_specs=[
            pl.BlockSpec((4, 2, RB, W144), lambda b: (0, 0, b, 0)),
            pl.BlockSpec((RB, 16), lambda b: (b, 0)),
            pl.BlockSpec((RB, D), lambda b: (b, 0)),
            pl.BlockSpec((RB, NETS), lambda b: (b, 0)),
            pl.BlockSpec((RB, NETS), lambda b: (b, 0)),
            pl.BlockSpec((RB, NETS), lambda b: (b, 0)),
            pl.BlockSpec((RB, NETS), lambda b: (b, 0)),
            full((OUT, D)),
            full((1, OUT)),
            full((OUT, D)),
            full((1, OUT)),
            full((OUT, D)),
            full((1, OUT)),
            full((1, D)),
            full((1, D)),
            full((1, 1)),
            full((1, 1)),
        ],
        out_specs=[
            pl.BlockSpec((RB, OUT), lambda b: (b, 0)),
            pl.BlockSpec((RB, 1), lambda b: (b, 0)),
            pl.BlockSpec((RB, 1), lambda b: (b, 0)),
            pl.BlockSpec((8, OUT), lambda b: (0, 0)),
        ],
        out_shape=[
            jax.ShapeDtypeStruct((N, OUT), jnp.float32),
            jax.ShapeDtypeStruct((N, 1), jnp.float32),
            jax.ShapeDtypeStruct((N, 1), jnp.float32),
            jax.ShapeDtypeStruct((8, OUT), jnp.float32),
        ],
    )(ox, aux, x, omt, ombt, imt, imbt,
      w_src, b_src, w_dst, b_dst, w_fc, b_fc, w_out_f, w_in_f,
      b_out_f, b_in_f)


def _bn_body(out_r, bn_r, g_r, b_r, y_r):
    mu = bn_r[0:1, :] * jnp.float32(1.0 / N)
    var = bn_r[1:2, :] * jnp.float32(1.0 / N) - mu * mu
    inv = lax.rsqrt(var + 1e-5)
    y_r[...] = (out_r[...] - mu) * inv * g_r[...] + b_r[...]


def _bn(out_pre, bn, gamma, beta):
    return pl.pallas_call(
        _bn_body,
        grid=(GB,),
        in_specs=[
            pl.BlockSpec((RB, OUT), lambda b: (b, 0)),
            pl.BlockSpec((8, OUT), lambda b: (0, 0)),
            pl.BlockSpec((1, OUT), lambda b: (0, 0)),
            pl.BlockSpec((1, OUT), lambda b: (0, 0)),
        ],
        out_specs=pl.BlockSpec((RB, OUT), lambda b: (b, 0)),
        out_shape=jax.ShapeDtypeStruct((N, OUT), jnp.float32),
    )(out_pre, bn, gamma, beta)


def kernel(x, edge_indices, edge_types, in_deg_idx, out_deg_idx,
           out_deg_mask, out_deg_mask_bias, in_deg_mask, in_deg_mask_bias,
           W_src, b_src, W_dst, b_dst, W_out_f, b_out_f, W_in_f, b_in_f,
           W_fc, b_fc, emb_in, emb_out, bn_gamma, bn_beta):
    ei = edge_indices.astype(jnp.int32)
    gsrc = jnp.stack([ei[0, 1], ei[0, 0], ei[1, 1], ei[1, 0]])
    gdst = jnp.stack([ei[0, 0], ei[0, 1], ei[1, 0], ei[1, 1]])
    gsrc = gsrc.reshape(4, NSUB, NB, BB)
    gdst = gdst.reshape(4, NSUB, NB, BB)
    z144 = jnp.zeros((RPT, W144), jnp.float32)

    tbl, aux = _prep(
        x, in_deg_idx.T, out_deg_idx.T,
        W_out_f.reshape(1, D), W_in_f.reshape(1, D), emb_in, emb_out)

    ox = _sc_spmm(tbl, gsrc, gdst, z144)

    out_pre, cin_m, cout_m, bn = _post(
        ox, aux, x,
        out_deg_mask.T, out_deg_mask_bias.T, in_deg_mask.T, in_deg_mask_bias.T,
        W_src, b_src.reshape(1, OUT), W_dst, b_dst.reshape(1, OUT),
        W_fc, b_fc.reshape(1, OUT), W_out_f.reshape(1, D),
        W_in_f.reshape(1, D), b_out_f.reshape(1, 1), b_in_f.reshape(1, 1))

    output = _bn(out_pre, bn, bn_gamma.reshape(1, OUT), bn_beta.reshape(1, OUT))
    return output, cin_m, cout_m


# speedup vs baseline: 1.1771x; 1.1771x over previous
import functools

import jax
import jax.numpy as jnp
from jax import lax
from jax.experimental import pallas as pl
from jax.experimental.pallas import tpu as pltpu
from jax.experimental.pallas import tpu_sc as plsc

N = 10000
D = 128
OUT = 128
E = 320000
NETS = 2
ALPHA = 0.5

NT = 32
NSUB = 16
PT = E // NT
BB = 100
NB = (E // NSUB) // BB
W144 = 144
NP = 10240
RPT = NP // NSUB
ET = E // NSUB
CH = 20
RB = 1000
GB = N // RB


def _sc_spmm(tbl, ei5, z144):
    mesh = plsc.VectorSubcoreMesh(core_axis_name="c", subcore_axis_name="s")
    half = NB // CH // 2

    @functools.partial(
        pl.kernel,
        out_type=jax.ShapeDtypeStruct((4, 2, NP, W144), jnp.float32),
        mesh=mesh,
        compiler_params=pltpu.CompilerParams(use_tc_tiling_on_sc=False),
        scratch_types=[
            pltpu.VMEM_SHARED((NP, W144), jnp.float32),
            pltpu.VMEM((CH, BB), jnp.int32),
            pltpu.VMEM((CH, BB), jnp.int32),
            pltpu.VMEM((CH, BB), jnp.int32),
            pltpu.VMEM((CH, BB), jnp.int32),
            pltpu.VMEM((BB, W144), jnp.float32),
            pltpu.VMEM((BB, W144), jnp.float32),
            pltpu.SemaphoreType.DMA,
            pltpu.SemaphoreType.DMA,
            pltpu.SemaphoreType.DMA,
            pltpu.SemaphoreType.DMA,
            pltpu.SemaphoreType.DMA,
            pltpu.SemaphoreType.DMA,
        ],
    )
    def k(tbl_h, ei_h, z144_h, out_h,
          acc, src_a, src_b, dst_a, dst_b, b0, b1,
          semg0, semg1, ss_a, sd_a, ss_b, sd_b):
        cid = lax.axis_index("c")
        sid = lax.axis_index("s")
        row0 = sid * RPT
        bufs = (b0, b1)
        semg = (semg0, semg1)
        for p in range(4):
            tblp = tbl_h.at[cid, p]
            gs = ei_h.at[p // 2, 1 - p % 2, sid]
            gd = ei_h.at[p // 2, p % 2, sid]
            pltpu.sync_copy(z144_h, acc.at[pl.ds(row0, RPT)])
            pltpu.sync_copy(gs.at[pl.ds(0, CH)], src_a)
            pltpu.sync_copy(gd.at[pl.ds(0, CH)], dst_a)
            pltpu.async_copy(gs.at[pl.ds(CH, CH)], src_b, ss_b)
            pltpu.async_copy(gd.at[pl.ds(CH, CH)], dst_b, sd_b)
            plsc.subcore_barrier()
            pltpu.async_copy(tblp.at[src_a.at[0]], b0, semg0)
            pltpu.async_copy(tblp.at[src_a.at[1]], b1, semg1)

            def body(q, c, tblp=tblp, gs=gs, gd=gd):
                @pl.when(q > 0)
                def _():
                    pltpu.make_async_copy(gd.at[pl.ds(0, CH)], dst_a,
                                          sd_a).wait()
                for j in range(CH):
                    par = j % 2
                    pltpu.make_async_copy(tblp.at[src_a.at[j]], bufs[par],
                                          semg[par]).wait()
                    pltpu.sync_copy(bufs[par], acc.at[dst_a.at[j]], add=True)
                    if j <= CH - 3:
                        pltpu.async_copy(tblp.at[src_a.at[j + 2]], bufs[par],
                                         semg[par])
                    elif j == CH - 2:
                        pltpu.make_async_copy(gs.at[pl.ds(0, CH)], src_b,
                                              ss_b).wait()
                        pltpu.async_copy(tblp.at[src_b.at[0]], bufs[par],
                                         semg[par])
                    else:
                        pltpu.async_copy(tblp.at[src_b.at[1]], bufs[par],
                                         semg[par])

                @pl.when(q < half - 1)
                def _():
                    pltpu.async_copy(gs.at[pl.ds((q + 1) * 2 * CH, CH)],
                                     src_a, ss_a)
                    pltpu.async_copy(gd.at[pl.ds((q + 1) * 2 * CH, CH)],
                                     dst_a, sd_a)

                pltpu.make_async_copy(gd.at[pl.ds(0, CH)], dst_b, sd_b).wait()
                for j in range(CH):
                    par = j % 2
                    pltpu.make_async_copy(tblp.at[src_b.at[j]], bufs[par],
                                          semg[par]).wait()
                    pltpu.sync_copy(bufs[par], acc.at[dst_b.at[j]], add=True)
                    if j <= CH - 3:
                        pltpu.async_copy(tblp.at[src_b.at[j + 2]], bufs[par],
                                         semg[par])
                    elif j == CH - 2:
                        @pl.when(q < half - 1)
                        def _(par=par):
                            pltpu.make_async_copy(gs.at[pl.ds(0, CH)], src_a,
                                                  ss_a).wait()
                            pltpu.async_copy(tblp.at[src_a.at[0]], bufs[par],
                                             semg[par])
                    else:
                        @pl.when(q < half - 1)
                        def _(par=par):
                            pltpu.async_copy(tblp.at[src_a.at[1]], bufs[par],
                                             semg[par])

                @pl.when(q < half - 1)
                def _():
                    pltpu.async_copy(gs.at[pl.ds((q + 1) * 2 * CH + CH, CH)],
                                     src_b, ss_b)
                    pltpu.async_copy(gd.at[pl.ds((q + 1) * 2 * CH + CH, CH)],
                                     dst_b, sd_b)
                return c

            lax.fori_loop(0, half, body, 0)
            plsc.subcore_barrier()
            pltpu.sync_copy(acc.at[pl.ds(row0, RPT)],
                            out_h.at[p, cid, pl.ds(row0, RPT)])

    return k(tbl, ei5, z144)


def _prep_body(x_r, id_r, od_r, wo_r, wi_r, ei_r, eo_r,
               tbl_r, aux_r):
    To = eo_r.shape[0]
    Ti = ei_r.shape[0]
    xb = x_r[...]
    r = 1.0 / (jnp.sum(jnp.abs(xb), axis=1, keepdims=True) + 1e-12)
    x1 = xb * r
    x1sq = x1 * x1
    wo = wo_r[...]
    wi = wi_r[...]
    s_out = jnp.sum(x1sq * wo, axis=1, keepdims=True)
    s_in = jnp.sum(x1sq * wi, axis=1, keepdims=True)
    embw_o = jnp.sum(eo_r[...] * wo, axis=1)
    embw_i = jnp.sum(ei_r[...] * wi, axis=1)
    z15 = jnp.zeros((RB, 15), jnp.float32)
    z16 = jnp.zeros((RB, 16), jnp.float32)
    cols = [s_out, s_in]
    invs = []
    for i in range(NETS):
        od = od_r[...][:, i]
        idg = id_r[...][:, i]
        oinv = jnp.where(od > 0,
                         lax.rsqrt(jnp.maximum(od.astype(jnp.float32), 1.0)),
                         0.0)
        iinv = jnp.where(idg > 0,
                         lax.rsqrt(jnp.maximum(idg.astype(jnp.float32), 1.0)),
                         0.0)
        ooh = od[:, None] == lax.broadcasted_iota(jnp.int32, (RB, To), 1)
        eo_g = jnp.sum(jnp.where(ooh, embw_o[None, :], 0.0), axis=1,
                       keepdims=True)
        ioh = idg[:, None] == lax.broadcasted_iota(jnp.int32, (RB, Ti), 1)
        ei_g = jnp.sum(jnp.where(ioh, embw_i[None, :], 0.0), axis=1,
                       keepdims=True)
        oinv = oinv[:, None]
        iinv = iinv[:, None]
        tbl_r[0, 2 * i] = jnp.concatenate([iinv * xb, iinv * s_out, z15],
                                          axis=1)
        tbl_r[0, 2 * i + 1] = jnp.concatenate([oinv * xb, oinv * s_in, z15],
                                              axis=1)
        tbl_r[1, 2 * i] = jnp.concatenate([iinv * x1, z16], axis=1)
        tbl_r[1, 2 * i + 1] = jnp.concatenate([oinv * x1, z16], axis=1)
        cols.append(eo_g)
        cols.append(ei_g)
        invs.append((oinv, iinv))
    aux_r[...] = jnp.concatenate(
        cols + [invs[0][0], invs[1][0], invs[0][1], invs[1][1],
                jnp.zeros((RB, 6), jnp.float32)], axis=1)


def _prep(x, ideg_t, odeg_t, w_out_f, w_in_f, emb_in, emb_out):
    To = emb_out.shape[0]
    Ti = emb_in.shape[0]
    return pl.pallas_call(
        _prep_body,
        grid=(GB,),
        in_specs=[
            pl.BlockSpec((RB, D), lambda b: (b, 0)),
            pl.BlockSpec((RB, NETS), lambda b: (b, 0)),
            pl.BlockSpec((RB, NETS), lambda b: (b, 0)),
            pl.BlockSpec((1, D), lambda b: (0, 0)),
            pl.BlockSpec((1, D), lambda b: (0, 0)),
            pl.BlockSpec((Ti, D), lambda b: (0, 0)),
            pl.BlockSpec((To, D), lambda b: (0, 0)),
        ],
        out_specs=[
            pl.BlockSpec((2, 4, RB, W144), lambda b: (0, 0, b, 0)),
            pl.BlockSpec((RB, 16), lambda b: (b, 0)),
        ],
        out_shape=[
            jax.ShapeDtypeStruct((2, 4, N, W144), jnp.float32),
            jax.ShapeDtypeStruct((N, 16), jnp.float32),
        ],
    )(x, ideg_t, odeg_t, w_out_f, w_in_f, emb_in, emb_out)


def _post_body(ox_r, aux_r, x_r, omt_r, ombt_r, imt_r, imbt_r,
               ws_r, bs_r, wd_r, bd_r, wf_r, bf_r, wo_r, wi_r, bof_r, bif_r,
               out_r, cin_r, cout_r, bn_r):
    pid = pl.program_id(0)
    xb = x_r[...]
    r = 1.0 / (jnp.sum(jnp.abs(xb), axis=1, keepdims=True) + 1e-12)
    x1 = xb * r
    aux = aux_r[...]
    gid = pid * RB + lax.broadcasted_iota(jnp.int32, (RB, 1), 0)
    dfac = jnp.where(gid == N - 1, 2.0, 1.0)
    wo = wo_r[...]
    wi = wi_r[...]
    tau = jnp.exp(jnp.float32(0.0)) + jnp.float32(0.1)
    acc_src = jnp.zeros((RB, D), jnp.float32)
    acc_dst = jnp.zeros((RB, D), jnp.float32)
    s_co = jnp.zeros((RB, 1), jnp.float32)
    s_ci = jnp.zeros((RB, 1), jnp.float32)
    for i in range(NETS):
        oinv = aux[:, 6 + i:7 + i]
        iinv = aux[:, 8 + i:9 + i]
        fa = ox_r[2 * i, 0]
        fb = ox_r[2 * i, 1]
        ga = ox_r[2 * i + 1, 0]
        gb = ox_r[2 * i + 1, 1]
        out_nei = fa[:, 0:D] * oinv
        mvs_o = fa[:, D:D + 1] * oinv
        mvx1_o = fb[:, 0:D] * oinv
        in_nei = ga[:, 0:D] * iinv
        mvs_i = ga[:, D:D + 1] * iinv
        mvx1_i = gb[:, 0:D] * iinv
        eow = (aux[:, 0:1] * dfac + mvs_o
               - 2.0 * jnp.sum(mvx1_o * x1 * wo, axis=1, keepdims=True))
        eiw = (aux[:, 1:2] * dfac + mvs_i
               - 2.0 * jnp.sum(mvx1_i * x1 * wi, axis=1, keepdims=True))
        co_l = -eow + aux[:, 2 + 2 * i:3 + 2 * i] + bof_r[0, 0]
        ci_l = -eiw + aux[:, 3 + 2 * i:4 + 2 * i] + bif_r[0, 0]
        co = 1.0 / (1.0 + jnp.exp((ci_l - co_l) / tau))
        ci = 1.0 - co
        co = co * omt_r[...][:, i:i + 1] + ombt_r[...][:, i:i + 1]
        ci = ci * imt_r[...][:, i:i + 1] + imbt_r[...][:, i:i + 1]
        acc_src += co * out_nei
        acc_dst += ci * in_nei
        s_co += co
        s_ci += ci
    dn = (((1,), (1,)), ((), ()))
    outv = (lax.dot_general(acc_src, ws_r[...], dn,
                            preferred_element_type=jnp.float32)
            + s_co * bs_r[...]
            + lax.dot_general(acc_dst, wd_r[...], dn,
                              preferred_element_type=jnp.float32)
            + s_ci * bd_r[...]) * jnp.float32(1.0 / NETS)
    outv = outv + ALPHA * (lax.dot_general(xb, wf_r[...], dn,
                                           preferred_element_type=jnp.float32)
                           + bf_r[...])
    out_r[...] = outv
    cout_r[...] = s_co * jnp.float32(1.0 / NETS)
    cin_r[...] = s_ci * jnp.float32(1.0 / NETS)

    @pl.when(pid == 0)
    def _():
        bn_r[...] = jnp.zeros((8, D), jnp.float32)

    bn_r[0:1, :] += jnp.sum(outv, axis=0, keepdims=True)
    bn_r[1:2, :] += jnp.sum(outv * outv, axis=0, keepdims=True)


def _post(ox, aux, x, omt, ombt, imt, imbt,
          w_src, b_src, w_dst, b_dst, w_fc, b_fc, w_out_f, w_in_f,
          b_out_f, b_in_f):
    full = lambda s: pl.BlockSpec(s, lambda b: tuple(0 for _ in s))
    return pl.pallas_call(
        _post_body,
        grid=(GB,),
        in_specs=[
            pl.BlockSpec((4, 2, RB, W144), lambda b: (0, 0, b, 0)),
            pl.BlockSpec((RB, 16), lambda b: (b, 0)),
            pl.BlockSpec((RB, D), lambda b: (b, 0)),
            pl.BlockSpec((RB, NETS), lambda b: (b, 0)),
            pl.BlockSpec((RB, NETS), lambda b: (b, 0)),
            pl.BlockSpec((RB, NETS), lambda b: (b, 0)),
            pl.BlockSpec((RB, NETS), lambda b: (b, 0)),
            full((OUT, D)),
            full((1, OUT)),
            full((OUT, D)),
            full((1, OUT)),
            full((OUT, D)),
            full((1, OUT)),
            full((1, D)),
            full((1, D)),
            full((1, 1)),
            full((1, 1)),
        ],
        out_specs=[
            pl.BlockSpec((RB, OUT), lambda b: (b, 0)),
            pl.BlockSpec((RB, 1), lambda b: (b, 0)),
            pl.BlockSpec((RB, 1), lambda b: (b, 0)),
            pl.BlockSpec((8, OUT), lambda b: (0, 0)),
        ],
        out_shape=[
            jax.ShapeDtypeStruct((N, OUT), jnp.float32),
            jax.ShapeDtypeStruct((N, 1), jnp.float32),
            jax.ShapeDtypeStruct((N, 1), jnp.float32),
            jax.ShapeDtypeStruct((8, OUT), jnp.float32),
        ],
    )(ox, aux, x, omt, ombt, imt, imbt,
      w_src, b_src, w_dst, b_dst, w_fc, b_fc, w_out_f, w_in_f,
      b_out_f, b_in_f)


def _bn_body(out_r, bn_r, g_r, b_r, y_r):
    mu = bn_r[0:1, :] * jnp.float32(1.0 / N)
    var = bn_r[1:2, :] * jnp.float32(1.0 / N) - mu * mu
    inv = lax.rsqrt(var + 1e-5)
    y_r[...] = (out_r[...] - mu) * inv * g_r[...] + b_r[...]


def _bn(out_pre, bn, gamma, beta):
    return pl.pallas_call(
        _bn_body,
        grid=(GB,),
        in_specs=[
            pl.BlockSpec((RB, OUT), lambda b: (b, 0)),
            pl.BlockSpec((8, OUT), lambda b: (0, 0)),
            pl.BlockSpec((1, OUT), lambda b: (0, 0)),
            pl.BlockSpec((1, OUT), lambda b: (0, 0)),
        ],
        out_specs=pl.BlockSpec((RB, OUT), lambda b: (b, 0)),
        out_shape=jax.ShapeDtypeStruct((N, OUT), jnp.float32),
    )(out_pre, bn, gamma, beta)


def kernel(x, edge_indices, edge_types, in_deg_idx, out_deg_idx,
           out_deg_mask, out_deg_mask_bias, in_deg_mask, in_deg_mask_bias,
           W_src, b_src, W_dst, b_dst, W_out_f, b_out_f, W_in_f, b_in_f,
           W_fc, b_fc, emb_in, emb_out, bn_gamma, bn_beta):
    ei5 = edge_indices.astype(jnp.int32).reshape(NETS, 2, NSUB, NB, BB)
    z144 = jnp.zeros((RPT, W144), jnp.float32)

    tbl, aux = _prep(
        x, in_deg_idx.T, out_deg_idx.T,
        W_out_f.reshape(1, D), W_in_f.reshape(1, D), emb_in, emb_out)

    ox = _sc_spmm(tbl, ei5, z144)

    out_pre, cin_m, cout_m, bn = _post(
        ox, aux, x,
        out_deg_mask.T, out_deg_mask_bias.T, in_deg_mask.T, in_deg_mask_bias.T,
        W_src, b_src.reshape(1, OUT), W_dst, b_dst.reshape(1, OUT),
        W_fc, b_fc.reshape(1, OUT), W_out_f.reshape(1, D),
        W_in_f.reshape(1, D), b_out_f.reshape(1, 1), b_in_f.reshape(1, 1))

    output = _bn(out_pre, bn, bn_gamma.reshape(1, OUT), bn_beta.reshape(1, OUT))
    return output, cin_m, cout_m


# speedup vs baseline: 1.1817x; 1.0040x over previous
import functools

import jax
import jax.numpy as jnp
from jax import lax
from jax.experimental import pallas as pl
from jax.experimental.pallas import tpu as pltpu
from jax.experimental.pallas import tpu_sc as plsc

N = 10000
D = 128
OUT = 128
E = 320000
NETS = 2
ALPHA = 0.5

NT = 32
NSUB = 16
PT = E // NT
BB = 100
NB = (E // NSUB) // BB
W144 = 144
NP = 10240
RPT = NP // NSUB
ET = E // NSUB
CH = 20
RB = 1000
GB = N // RB


def _sc_spmm(tbl, ei5, z144):
    mesh = plsc.VectorSubcoreMesh(core_axis_name="c", subcore_axis_name="s")
    half = NB // CH // 2

    @functools.partial(
        pl.kernel,
        out_type=jax.ShapeDtypeStruct((4, 2, NP, W144), jnp.float32),
        mesh=mesh,
        compiler_params=pltpu.CompilerParams(use_tc_tiling_on_sc=False),
        scratch_types=[
            pltpu.VMEM_SHARED((NP, W144), jnp.float32),
            pltpu.VMEM((CH, BB), jnp.int32),
            pltpu.VMEM((CH, BB), jnp.int32),
            pltpu.VMEM((CH, BB), jnp.int32),
            pltpu.VMEM((CH, BB), jnp.int32),
            pltpu.VMEM((BB, W144), jnp.float32),
            pltpu.VMEM((BB, W144), jnp.float32),
            pltpu.SemaphoreType.DMA,
            pltpu.SemaphoreType.DMA,
            pltpu.SemaphoreType.DMA,
            pltpu.SemaphoreType.DMA,
            pltpu.SemaphoreType.DMA,
            pltpu.SemaphoreType.DMA,
        ],
    )
    def k(tbl_h, ei_h, z144_h, out_h,
          acc, src_a, src_b, dst_a, dst_b, b0, b1,
          semg0, semg1, ss_a, sd_a, ss_b, sd_b):
        cid = lax.axis_index("c")
        sid = lax.axis_index("s")
        row0 = sid * RPT
        bufs = (b0, b1)
        semg = (semg0, semg1)
        for p in range(4):
            tblp = tbl_h.at[cid, p]
            gs = ei_h.at[p // 2, 1 - p % 2, sid]
            gd = ei_h.at[p // 2, p % 2, sid]
            pltpu.sync_copy(z144_h, b0.at[pl.ds(0, 80)])
            for kz in range(RPT // 80):
                pltpu.sync_copy(b0.at[pl.ds(0, 80)],
                                acc.at[pl.ds(row0 + kz * 80, 80)])
            pltpu.sync_copy(gs.at[pl.ds(0, CH)], src_a)
            pltpu.sync_copy(gd.at[pl.ds(0, CH)], dst_a)
            pltpu.async_copy(gs.at[pl.ds(CH, CH)], src_b, ss_b)
            pltpu.async_copy(gd.at[pl.ds(CH, CH)], dst_b, sd_b)
            plsc.subcore_barrier()
            pltpu.async_copy(tblp.at[src_a.at[0]], b0, semg0)
            pltpu.async_copy(tblp.at[src_a.at[1]], b1, semg1)

            def body(q, c, tblp=tblp, gs=gs, gd=gd):
                @pl.when(q > 0)
                def _():
                    pltpu.make_async_copy(gd.at[pl.ds(0, CH)], dst_a,
                                          sd_a).wait()
                for j in range(CH):
                    par = j % 2
                    pltpu.make_async_copy(tblp.at[src_a.at[j]], bufs[par],
                                          semg[par]).wait()
                    pltpu.sync_copy(bufs[par], acc.at[dst_a.at[j]], add=True)
                    if j <= CH - 3:
                        pltpu.async_copy(tblp.at[src_a.at[j + 2]], bufs[par],
                                         semg[par])
                    elif j == CH - 2:
                        pltpu.make_async_copy(gs.at[pl.ds(0, CH)], src_b,
                                              ss_b).wait()
                        pltpu.async_copy(tblp.at[src_b.at[0]], bufs[par],
                                         semg[par])
                    else:
                        pltpu.async_copy(tblp.at[src_b.at[1]], bufs[par],
                                         semg[par])

                @pl.when(q < half - 1)
                def _():
                    pltpu.async_copy(gs.at[pl.ds((q + 1) * 2 * CH, CH)],
                                     src_a, ss_a)
                    pltpu.async_copy(gd.at[pl.ds((q + 1) * 2 * CH, CH)],
                                     dst_a, sd_a)

                pltpu.make_async_copy(gd.at[pl.ds(0, CH)], dst_b, sd_b).wait()
                for j in range(CH):
                    par = j % 2
                    pltpu.make_async_copy(tblp.at[src_b.at[j]], bufs[par],
                                          semg[par]).wait()
                    pltpu.sync_copy(bufs[par], acc.at[dst_b.at[j]], add=True)
                    if j <= CH - 3:
                        pltpu.async_copy(tblp.at[src_b.at[j + 2]], bufs[par],
                                         semg[par])
                    elif j == CH - 2:
                        @pl.when(q < half - 1)
                        def _(par=par):
                            pltpu.make_async_copy(gs.at[pl.ds(0, CH)], src_a,
                                                  ss_a).wait()
                            pltpu.async_copy(tblp.at[src_a.at[0]], bufs[par],
                                             semg[par])
                    else:
                        @pl.when(q < half - 1)
                        def _(par=par):
                            pltpu.async_copy(tblp.at[src_a.at[1]], bufs[par],
                                             semg[par])

                @pl.when(q < half - 1)
                def _():
                    pltpu.async_copy(gs.at[pl.ds((q + 1) * 2 * CH + CH, CH)],
                                     src_b, ss_b)
                    pltpu.async_copy(gd.at[pl.ds((q + 1) * 2 * CH + CH, CH)],
                                     dst_b, sd_b)
                return c

            lax.fori_loop(0, half, body, 0)
            plsc.subcore_barrier()
            pltpu.sync_copy(acc.at[pl.ds(row0, RPT)],
                            out_h.at[p, cid, pl.ds(row0, RPT)])

    return k(tbl, ei5, z144)


def _prep_body(x_r, id_r, od_r, wo_r, wi_r, ei_r, eo_r,
               tbl_r, aux_r):
    To = eo_r.shape[0]
    Ti = ei_r.shape[0]
    xb = x_r[...]
    r = 1.0 / (jnp.sum(jnp.abs(xb), axis=1, keepdims=True) + 1e-12)
    x1 = xb * r
    x1sq = x1 * x1
    wo = wo_r[...]
    wi = wi_r[...]
    s_out = jnp.sum(x1sq * wo, axis=1, keepdims=True)
    s_in = jnp.sum(x1sq * wi, axis=1, keepdims=True)
    embw_o = jnp.sum(eo_r[...] * wo, axis=1)
    embw_i = jnp.sum(ei_r[...] * wi, axis=1)
    z15 = jnp.zeros((RB, 15), jnp.float32)
    z16 = jnp.zeros((RB, 16), jnp.float32)
    cols = [s_out, s_in]
    invs = []
    for i in range(NETS):
        od = od_r[...][:, i]
        idg = id_r[...][:, i]
        oinv = jnp.where(od > 0,
                         lax.rsqrt(jnp.maximum(od.astype(jnp.float32), 1.0)),
                         0.0)
        iinv = jnp.where(idg > 0,
                         lax.rsqrt(jnp.maximum(idg.astype(jnp.float32), 1.0)),
                         0.0)
        ooh = od[:, None] == lax.broadcasted_iota(jnp.int32, (RB, To), 1)
        eo_g = jnp.sum(jnp.where(ooh, embw_o[None, :], 0.0), axis=1,
                       keepdims=True)
        ioh = idg[:, None] == lax.broadcasted_iota(jnp.int32, (RB, Ti), 1)
        ei_g = jnp.sum(jnp.where(ioh, embw_i[None, :], 0.0), axis=1,
                       keepdims=True)
        oinv = oinv[:, None]
        iinv = iinv[:, None]
        tbl_r[0, 2 * i] = jnp.concatenate([iinv * xb, iinv * s_out, z15],
                                          axis=1)
        tbl_r[0, 2 * i + 1] = jnp.concatenate([oinv * xb, oinv * s_in, z15],
                                              axis=1)
        tbl_r[1, 2 * i] = jnp.concatenate([iinv * x1, z16], axis=1)
        tbl_r[1, 2 * i + 1] = jnp.concatenate([oinv * x1, z16], axis=1)
        cols.append(eo_g)
        cols.append(ei_g)
        invs.append((oinv, iinv))
    aux_r[...] = jnp.concatenate(
        cols + [invs[0][0], invs[1][0], invs[0][1], invs[1][1],
                jnp.zeros((RB, 6), jnp.float32)], axis=1)


def _prep(x, ideg_t, odeg_t, w_out_f, w_in_f, emb_in, emb_out):
    To = emb_out.shape[0]
    Ti = emb_in.shape[0]
    return pl.pallas_call(
        _prep_body,
        grid=(GB,),
        in_specs=[
            pl.BlockSpec((RB, D), lambda b: (b, 0)),
            pl.BlockSpec((RB, NETS), lambda b: (b, 0)),
            pl.BlockSpec((RB, NETS), lambda b: (b, 0)),
            pl.BlockSpec((1, D), lambda b: (0, 0)),
            pl.BlockSpec((1, D), lambda b: (0, 0)),
            pl.BlockSpec((Ti, D), lambda b: (0, 0)),
            pl.BlockSpec((To, D), lambda b: (0, 0)),
        ],
        out_specs=[
            pl.BlockSpec((2, 4, RB, W144), lambda b: (0, 0, b, 0)),
            pl.BlockSpec((RB, 16), lambda b: (b, 0)),
        ],
        out_shape=[
            jax.ShapeDtypeStruct((2, 4, N, W144), jnp.float32),
            jax.ShapeDtypeStruct((N, 16), jnp.float32),
        ],
    )(x, ideg_t, odeg_t, w_out_f, w_in_f, emb_in, emb_out)


def _post_body(ox_r, aux_r, x_r, omt_r, ombt_r, imt_r, imbt_r,
               ws_r, bs_r, wd_r, bd_r, wf_r, bf_r, wo_r, wi_r, bof_r, bif_r,
               out_r, cin_r, cout_r, bn_r):
    pid = pl.program_id(0)
    xb = x_r[...]
    r = 1.0 / (jnp.sum(jnp.abs(xb), axis=1, keepdims=True) + 1e-12)
    x1 = xb * r
    aux = aux_r[...]
    gid = pid * RB + lax.broadcasted_iota(jnp.int32, (RB, 1), 0)
    dfac = jnp.where(gid == N - 1, 2.0, 1.0)
    wo = wo_r[...]
    wi = wi_r[...]
    tau = jnp.exp(jnp.float32(0.0)) + jnp.float32(0.1)
    acc_src = jnp.zeros((RB, D), jnp.float32)
    acc_dst = jnp.zeros((RB, D), jnp.float32)
    s_co = jnp.zeros((RB, 1), jnp.float32)
    s_ci = jnp.zeros((RB, 1), jnp.float32)
    for i in range(NETS):
        oinv = aux[:, 6 + i:7 + i]
        iinv = aux[:, 8 + i:9 + i]
        fa = ox_r[2 * i, 0]
        fb = ox_r[2 * i, 1]
        ga = ox_r[2 * i + 1, 0]
        gb = ox_r[2 * i + 1, 1]
        out_nei = fa[:, 0:D] * oinv
        mvs_o = fa[:, D:D + 1] * oinv
        mvx1_o = fb[:, 0:D] * oinv
        in_nei = ga[:, 0:D] * iinv
        mvs_i = ga[:, D:D + 1] * iinv
        mvx1_i = gb[:, 0:D] * iinv
        eow = (aux[:, 0:1] * dfac + mvs_o
               - 2.0 * jnp.sum(mvx1_o * x1 * wo, axis=1, keepdims=True))
        eiw = (aux[:, 1:2] * dfac + mvs_i
               - 2.0 * jnp.sum(mvx1_i * x1 * wi, axis=1, keepdims=True))
        co_l = -eow + aux[:, 2 + 2 * i:3 + 2 * i] + bof_r[0, 0]
        ci_l = -eiw + aux[:, 3 + 2 * i:4 + 2 * i] + bif_r[0, 0]
        co = 1.0 / (1.0 + jnp.exp((ci_l - co_l) / tau))
        ci = 1.0 - co
        co = co * omt_r[...][:, i:i + 1] + ombt_r[...][:, i:i + 1]
        ci = ci * imt_r[...][:, i:i + 1] + imbt_r[...][:, i:i + 1]
        acc_src += co * out_nei
        acc_dst += ci * in_nei
        s_co += co
        s_ci += ci
    dn = (((1,), (1,)), ((), ()))
    outv = (lax.dot_general(acc_src, ws_r[...], dn,
                            preferred_element_type=jnp.float32)
            + s_co * bs_r[...]
            + lax.dot_general(acc_dst, wd_r[...], dn,
                              preferred_element_type=jnp.float32)
            + s_ci * bd_r[...]) * jnp.float32(1.0 / NETS)
    outv = outv + ALPHA * (lax.dot_general(xb, wf_r[...], dn,
                                           preferred_element_type=jnp.float32)
                           + bf_r[...])
    out_r[...] = outv
    cout_r[...] = s_co * jnp.float32(1.0 / NETS)
    cin_r[...] = s_ci * jnp.float32(1.0 / NETS)

    @pl.when(pid == 0)
    def _():
        bn_r[...] = jnp.zeros((8, D), jnp.float32)

    bn_r[0:1, :] += jnp.sum(outv, axis=0, keepdims=True)
    bn_r[1:2, :] += jnp.sum(outv * outv, axis=0, keepdims=True)


def _post(ox, aux, x, omt, ombt, imt, imbt,
          w_src, b_src, w_dst, b_dst, w_fc, b_fc, w_out_f, w_in_f,
          b_out_f, b_in_f):
    full = lambda s: pl.BlockSpec(s, lambda b: tuple(0 for _ in s))
    return pl.pallas_call(
        _post_body,
        grid=(GB,),
        in_specs=[
            pl.BlockSpec((4, 2, RB, W144), lambda b: (0, 0, b, 0)),
            pl.BlockSpec((RB, 16), lambda b: (b, 0)),
            pl.BlockSpec((RB, D), lambda b: (b, 0)),
            pl.BlockSpec((RB, NETS), lambda b: (b, 0)),
            pl.BlockSpec((RB, NETS), lambda b: (b, 0)),
            pl.BlockSpec((RB, NETS), lambda b: (b, 0)),
            pl.BlockSpec((RB, NETS), lambda b: (b, 0)),
            full((OUT, D)),
            full((1, OUT)),
            full((OUT, D)),
            full((1, OUT)),
            full((OUT, D)),
            full((1, OUT)),
            full((1, D)),
            full((1, D)),
            full((1, 1)),
            full((1, 1)),
        ],
        out_specs=[
            pl.BlockSpec((RB, OUT), lambda b: (b, 0)),
            pl.BlockSpec((RB, 1), lambda b: (b, 0)),
            pl.BlockSpec((RB, 1), lambda b: (b, 0)),
            pl.BlockSpec((8, OUT), lambda b: (0, 0)),
        ],
        out_shape=[
            jax.ShapeDtypeStruct((N, OUT), jnp.float32),
            jax.ShapeDtypeStruct((N, 1), jnp.float32),
            jax.ShapeDtypeStruct((N, 1), jnp.float32),
            jax.ShapeDtypeStruct((8, OUT), jnp.float32),
        ],
    )(ox, aux, x, omt, ombt, imt, imbt,
      w_src, b_src, w_dst, b_dst, w_fc, b_fc, w_out_f, w_in_f,
      b_out_f, b_in_f)


def _bn_body(out_r, bn_r, g_r, b_r, y_r):
    mu = bn_r[0:1, :] * jnp.float32(1.0 / N)
    var = bn_r[1:2, :] * jnp.float32(1.0 / N) - mu * mu
    inv = lax.rsqrt(var + 1e-5)
    y_r[...] = (out_r[...] - mu) * inv * g_r[...] + b_r[...]


def _bn(out_pre, bn, gamma, beta):
    return pl.pallas_call(
        _bn_body,
        grid=(GB,),
        in_specs=[
            pl.BlockSpec((RB, OUT), lambda b: (b, 0)),
            pl.BlockSpec((8, OUT), lambda b: (0, 0)),
            pl.BlockSpec((1, OUT), lambda b: (0, 0)),
            pl.BlockSpec((1, OUT), lambda b: (0, 0)),
        ],
        out_specs=pl.BlockSpec((RB, OUT), lambda b: (b, 0)),
        out_shape=jax.ShapeDtypeStruct((N, OUT), jnp.float32),
    )(out_pre, bn, gamma, beta)


def kernel(x, edge_indices, edge_types, in_deg_idx, out_deg_idx,
           out_deg_mask, out_deg_mask_bias, in_deg_mask, in_deg_mask_bias,
           W_src, b_src, W_dst, b_dst, W_out_f, b_out_f, W_in_f, b_in_f,
           W_fc, b_fc, emb_in, emb_out, bn_gamma, bn_beta):
    ei5 = edge_indices.astype(jnp.int32).reshape(NETS, 2, NSUB, NB, BB)
    z144 = jnp.zeros((80, W144), jnp.float32)

    tbl, aux = _prep(
        x, in_deg_idx.T, out_deg_idx.T,
        W_out_f.reshape(1, D), W_in_f.reshape(1, D), emb_in, emb_out)

    ox = _sc_spmm(tbl, ei5, z144)

    out_pre, cin_m, cout_m, bn = _post(
        ox, aux, x,
        out_deg_mask.T, out_deg_mask_bias.T, in_deg_mask.T, in_deg_mask_bias.T,
        W_src, b_src.reshape(1, OUT), W_dst, b_dst.reshape(1, OUT),
        W_fc, b_fc.reshape(1, OUT), W_out_f.reshape(1, D),
        W_in_f.reshape(1, D), b_out_f.reshape(1, 1), b_in_f.reshape(1, 1))

    output = _bn(out_pre, bn, bn_gamma.reshape(1, OUT), bn_beta.reshape(1, OUT))
    return output, cin_m, cout_m
